# baseline (speedup 1.0000x reference)
import functools

import jax
import jax.numpy as jnp
import numpy as np
from jax import lax
from jax.experimental import pallas as pl
from jax.experimental.pallas import tpu as pltpu
from jax.experimental.pallas import tpu_sc as plsc

_TP = 256
_NC = 2
_NS = 16
_NW = _NC * _NS
_CAP = 48
_I32MIN = np.int32(-2147483648)


def _thresh_body(n_pass, a_ref, t_ref):
    a = a_ref[0]
    bits = lax.bitcast_convert_type(a, jnp.int32)
    ks = jnp.where(bits < 0, _I32MIN - bits, bits)
    ku = lax.bitcast_convert_type(ks ^ _I32MIN, jnp.uint32)
    prefix = jnp.zeros((1, a.shape[1]), jnp.uint32)
    for bit in range(31, -1, -1):
        cand = prefix | np.uint32(1 << bit)
        c = jnp.sum((ku >= cand).astype(jnp.int32), axis=0, keepdims=True)
        prefix = jnp.where(c >= n_pass, cand, prefix)
    ks_t = lax.bitcast_convert_type(prefix, jnp.int32) ^ _I32MIN
    bits_t = jnp.where(ks_t < 0, _I32MIN - ks_t, ks_t)
    t_ref[0] = lax.bitcast_convert_type(bits_t, jnp.float32)


def _thresholds(a3, n_pass):
    B, F, P = a3.shape
    tp = min(_TP, P)
    t = pl.pallas_call(
        functools.partial(_thresh_body, n_pass),
        grid=(B, P // tp),
        in_specs=[pl.BlockSpec((1, F, tp), lambda b, p: (b, 0, p))],
        out_specs=pl.BlockSpec((1, 1, tp), lambda b, p: (b, 0, p)),
        out_shape=jax.ShapeDtypeStruct((B, 1, P), jnp.float32),
    )(a3)
    return t.reshape(B * P)


def _sc_rank_encode(act2, t_flat, wext, B, F, P, n_pass):
    n_ext = n_pass + 1
    n_chunks_total = (B * P) // 16
    chunks_per_w = n_chunks_total // _NW
    mesh = plsc.VectorSubcoreMesh(core_axis_name="c", subcore_axis_name="s")

    @functools.partial(
        pl.kernel,
        out_type=jax.ShapeDtypeStruct((B * F * P,), jnp.float32),
        mesh=mesh,
        compiler_params=pltpu.CompilerParams(needs_layout_passes=False),
        scratch_types=[
            pltpu.VMEM((F * n_ext,), jnp.float32),
            pltpu.VMEM((F * 16,), jnp.float32),
            pltpu.VMEM((F * 16,), jnp.float32),
            pltpu.VMEM((16,), jnp.float32),
            pltpu.VMEM((16, 64), jnp.float32),
            pltpu.VMEM((16, 64), jnp.int32),
        ],
    )
    def sck(act_hbm, t_hbm, wext_hbm, out_hbm, w_v, a_v, o_v, t_v, cv_v, cc_v):
        wid = lax.axis_index("s") * _NC + lax.axis_index("c")
        pltpu.sync_copy(wext_hbm, w_v)
        iota = lax.iota(jnp.int32, 16)
        neg_inf = jnp.full((16,), -jnp.inf, jnp.float32)
        zeros16 = jnp.zeros((16,), jnp.float32)

        def chunk_body(ci, _):
            chunk = wid * chunks_per_w + ci
            pltpu.sync_copy(
                act_hbm.at[pl.ds(chunk * (F * 16), F * 16)], a_v
            )
            pltpu.sync_copy(t_hbm.at[pl.ds(chunk * 16, 16)], t_v)
            t_vec = t_v[...]

            def init_body(pp, _):
                for j in range(4):
                    cv_v[pp, pl.ds(j * 16, 16)] = neg_inf
                return _

            lax.fori_loop(0, 16, init_body, 0)

            def scan_body(f, cnt):
                v = a_v[pl.ds(f * 16, 16)]
                m = v >= t_vec
                slot = jnp.minimum(cnt, _CAP)
                plsc.store_scatter(cv_v, [iota, slot], v, mask=m)
                plsc.store_scatter(
                    cc_v, [iota, slot], jnp.full((16,), f, jnp.int32), mask=m
                )
                o_v[pl.ds(f * 16, 16)] = zeros16
                return cnt + m.astype(jnp.int32)

            lax.fori_loop(0, F, scan_body, jnp.zeros((16,), jnp.int32))

            def pos_body(pp, carry):
                k1 = cv_v[pp, pl.ds(0, 16)]
                c1 = cc_v[pp, pl.ds(0, 16)]
                k2 = cv_v[pp, pl.ds(16, 16)]
                c2 = cc_v[pp, pl.ds(16, 16)]
                k3 = cv_v[pp, pl.ds(32, 16)]
                c3 = cc_v[pp, pl.ds(32, 16)]
                k1, c1 = plsc.sort_key_val(k1, c1, descending=True)
                k2, c2 = plsc.sort_key_val(k2, c2, descending=True)
                k3, c3 = plsc.sort_key_val(k3, c3, descending=True)
                k2r = lax.rev(k2, (0,))
                c2r = lax.rev(c2, (0,))
                m12 = k1 >= k2r
                hi_k = jnp.where(m12, k1, k2r)
                hi_c = jnp.where(m12, c1, c2r)
                lo_k = jnp.where(m12, k2r, k1)
                lo_c = jnp.where(m12, c2r, c1)
                hi_k, hi_c = plsc.sort_key_val(hi_k, hi_c, descending=True)
                lo_k, lo_c = plsc.sort_key_val(lo_k, lo_c, descending=True)
                k3r = lax.rev(k3, (0,))
                c3r = lax.rev(c3, (0,))
                m3 = lo_k >= k3r
                x2_k = jnp.where(m3, lo_k, k3r)
                x2_c = jnp.where(m3, lo_c, c3r)
                mh = hi_k >= x2_k
                yh_k = jnp.where(mh, hi_k, x2_k)
                yh_c = jnp.where(mh, hi_c, x2_c)
                yl_k = jnp.where(mh, x2_k, hi_k)
                yl_c = jnp.where(mh, x2_c, hi_c)
                yh_ks, yh_c = plsc.sort_key_val(yh_k, yh_c, descending=True)
                del yh_ks
                yl_ks, yl_c = plsc.sort_key_val(yl_k, yl_c, descending=True)
                del yl_ks
                w_hi = plsc.load_gather(w_v, [yh_c * n_ext + iota])
                w_lo = plsc.load_gather(w_v, [yl_c * n_ext + (iota + 16)])
                pvec = jnp.full((16,), pp, jnp.int32)
                plsc.store_scatter(o_v, [yh_c * 16 + pvec], w_hi)
                plsc.store_scatter(o_v, [yl_c * 16 + pvec], w_lo)
                return carry

            lax.fori_loop(0, 16, pos_body, 0)
            pltpu.sync_copy(
                o_v, out_hbm.at[pl.ds(chunk * (F * 16), F * 16)]
            )
            return _

        lax.fori_loop(0, chunks_per_w, chunk_body, 0)

    return sck(act2, t_flat, wext)


def kernel(activations, rank_weights):
    B, F, H, W = activations.shape
    n_pass = rank_weights.shape[1]
    P = H * W
    a3 = activations.reshape(B, F, P)
    t_flat = _thresholds(a3, n_pass)
    wext = jnp.concatenate(
        [rank_weights, jnp.zeros((F, 1), jnp.float32)], axis=1
    ).reshape(-1)
    act_r = jnp.transpose(
        activations.reshape(B, F, P // 16, 16), (0, 2, 1, 3)
    ).reshape(-1)
    out_r = _sc_rank_encode(act_r, t_flat, wext, B, F, P, n_pass)
    out = jnp.transpose(
        out_r.reshape(B, P // 16, F, 16), (0, 2, 1, 3)
    )
    return out.reshape(B, F, H, W)


# speedup vs baseline: 1.0017x; 1.0017x over previous
import functools

import jax
import jax.numpy as jnp
import numpy as np
from jax import lax
from jax.experimental import pallas as pl
from jax.experimental.pallas import tpu as pltpu
from jax.experimental.pallas import tpu_sc as plsc

_TP = 256
_NC = 2
_NS = 16
_NW = _NC * _NS
_CAP = 48
_I32MIN = np.int32(-2147483648)


def _thresh_body(n_pass, a_ref, t_ref):
    a = a_ref[0]
    bits = lax.bitcast_convert_type(a, jnp.int32)
    ks = jnp.where(bits < 0, _I32MIN - bits, bits)
    ku = lax.bitcast_convert_type(ks ^ _I32MIN, jnp.uint32)
    prefix = jnp.zeros((1, a.shape[1]), jnp.uint32)
    for bit in range(31, -1, -1):
        cand = prefix | np.uint32(1 << bit)
        c = jnp.sum((ku >= cand).astype(jnp.int32), axis=0, keepdims=True)
        prefix = jnp.where(c >= n_pass, cand, prefix)
    ks_t = lax.bitcast_convert_type(prefix, jnp.int32) ^ _I32MIN
    bits_t = jnp.where(ks_t < 0, _I32MIN - ks_t, ks_t)
    t_ref[0] = lax.bitcast_convert_type(bits_t, jnp.float32)


def _thresholds(a3, n_pass):
    B, F, P = a3.shape
    tp = min(_TP, P)
    t = pl.pallas_call(
        functools.partial(_thresh_body, n_pass),
        grid=(B, P // tp),
        in_specs=[pl.BlockSpec((1, F, tp), lambda b, p: (b, 0, p))],
        out_specs=pl.BlockSpec((1, 1, tp), lambda b, p: (b, 0, p)),
        out_shape=jax.ShapeDtypeStruct((B, 1, P), jnp.float32),
    )(a3)
    return t.reshape(B * P)


def _sc_rank_encode(act2, t_flat, wext, B, F, P, n_pass):
    n_ext = n_pass + 1
    n_chunks_total = (B * P) // 16
    chunks_per_w = n_chunks_total // _NW
    mesh = plsc.VectorSubcoreMesh(core_axis_name="c", subcore_axis_name="s")

    @functools.partial(
        pl.kernel,
        out_type=jax.ShapeDtypeStruct((B * F * P,), jnp.float32),
        mesh=mesh,
        compiler_params=pltpu.CompilerParams(needs_layout_passes=False),
        scratch_types=[
            pltpu.VMEM((F * n_ext,), jnp.float32),
            pltpu.VMEM((F * 16,), jnp.float32),
            pltpu.VMEM((F * 16,), jnp.float32),
            pltpu.VMEM((16,), jnp.float32),
            pltpu.VMEM((16, 64), jnp.float32),
            pltpu.VMEM((16, 64), jnp.int32),
        ],
    )
    def sck(act_hbm, t_hbm, wext_hbm, out_hbm, w_v, a_v, o_v, t_v, cv_v, cc_v):
        wid = lax.axis_index("s") * _NC + lax.axis_index("c")
        pltpu.sync_copy(wext_hbm, w_v)
        iota = lax.iota(jnp.int32, 16)
        neg_inf = jnp.full((16,), -jnp.inf, jnp.float32)
        zeros16 = jnp.zeros((16,), jnp.float32)

        def chunk_body(ci, _):
            chunk = wid * chunks_per_w + ci
            pltpu.sync_copy(
                act_hbm.at[pl.ds(chunk * (F * 16), F * 16)], a_v
            )
            pltpu.sync_copy(t_hbm.at[pl.ds(chunk * 16, 16)], t_v)
            t_vec = t_v[...]

            def init_body(pp, _):
                for j in range(4):
                    cv_v[pp, pl.ds(j * 16, 16)] = neg_inf
                return _

            lax.fori_loop(0, 16, init_body, 0, unroll=4)

            def scan_body(f, cnt):
                v = a_v[pl.ds(f * 16, 16)]
                m = v >= t_vec
                slot = jnp.minimum(cnt, _CAP)
                plsc.store_scatter(cv_v, [iota, slot], v, mask=m)
                plsc.store_scatter(
                    cc_v, [iota, slot], jnp.full((16,), f, jnp.int32), mask=m
                )
                o_v[pl.ds(f * 16, 16)] = zeros16
                return cnt + m.astype(jnp.int32)

            lax.fori_loop(0, F, scan_body, jnp.zeros((16,), jnp.int32), unroll=8)

            def pos_body(pp, carry):
                k1 = cv_v[pp, pl.ds(0, 16)]
                c1 = cc_v[pp, pl.ds(0, 16)]
                k2 = cv_v[pp, pl.ds(16, 16)]
                c2 = cc_v[pp, pl.ds(16, 16)]
                k3 = cv_v[pp, pl.ds(32, 16)]
                c3 = cc_v[pp, pl.ds(32, 16)]
                k1, c1 = plsc.sort_key_val(k1, c1, descending=True)
                k2, c2 = plsc.sort_key_val(k2, c2, descending=True)
                k3, c3 = plsc.sort_key_val(k3, c3, descending=True)
                k2r = lax.rev(k2, (0,))
                c2r = lax.rev(c2, (0,))
                m12 = k1 >= k2r
                hi_k = jnp.where(m12, k1, k2r)
                hi_c = jnp.where(m12, c1, c2r)
                lo_k = jnp.where(m12, k2r, k1)
                lo_c = jnp.where(m12, c2r, c1)
                hi_k, hi_c = plsc.sort_key_val(hi_k, hi_c, descending=True)
                lo_k, lo_c = plsc.sort_key_val(lo_k, lo_c, descending=True)
                k3r = lax.rev(k3, (0,))
                c3r = lax.rev(c3, (0,))
                m3 = lo_k >= k3r
                x2_k = jnp.where(m3, lo_k, k3r)
                x2_c = jnp.where(m3, lo_c, c3r)
                mh = hi_k >= x2_k
                yh_k = jnp.where(mh, hi_k, x2_k)
                yh_c = jnp.where(mh, hi_c, x2_c)
                yl_k = jnp.where(mh, x2_k, hi_k)
                yl_c = jnp.where(mh, x2_c, hi_c)
                yh_ks, yh_c = plsc.sort_key_val(yh_k, yh_c, descending=True)
                del yh_ks
                yl_ks, yl_c = plsc.sort_key_val(yl_k, yl_c, descending=True)
                del yl_ks
                w_hi = plsc.load_gather(w_v, [yh_c * n_ext + iota])
                w_lo = plsc.load_gather(w_v, [yl_c * n_ext + (iota + 16)])
                pvec = jnp.full((16,), pp, jnp.int32)
                plsc.store_scatter(o_v, [yh_c * 16 + pvec], w_hi)
                plsc.store_scatter(o_v, [yl_c * 16 + pvec], w_lo)
                return carry

            lax.fori_loop(0, 16, pos_body, 0, unroll=2)
            pltpu.sync_copy(
                o_v, out_hbm.at[pl.ds(chunk * (F * 16), F * 16)]
            )
            return _

        lax.fori_loop(0, chunks_per_w, chunk_body, 0)

    return sck(act2, t_flat, wext)


def kernel(activations, rank_weights):
    B, F, H, W = activations.shape
    n_pass = rank_weights.shape[1]
    P = H * W
    a3 = activations.reshape(B, F, P)
    t_flat = _thresholds(a3, n_pass)
    wext = jnp.concatenate(
        [rank_weights, jnp.zeros((F, 1), jnp.float32)], axis=1
    ).reshape(-1)
    act_r = jnp.transpose(
        activations.reshape(B, F, P // 16, 16), (0, 2, 1, 3)
    ).reshape(-1)
    out_r = _sc_rank_encode(act_r, t_flat, wext, B, F, P, n_pass)
    out = jnp.transpose(
        out_r.reshape(B, P // 16, F, 16), (0, 2, 1, 3)
    )
    return out.reshape(B, F, H, W)
